# baseline (speedup 1.0000x reference)
import functools

import jax
import jax.numpy as jnp
from jax import lax
from jax.experimental import pallas as pl
from jax.experimental.pallas import tpu as pltpu
from jax.experimental.pallas import tpu_sc as plsc

B_ROWS = 4096
S_COLS = 50
SP = 56
D = 110
DP = 128
V = 96

A_TC = 2048
B_SC = B_ROWS - A_TC

NC = 2
NS = 16
NW = NC * NS
B_PER_W = B_SC // NW
NB = 8
N_STEPS = B_PER_W // NB
IDX_PER_STEP = NB * SP
IDX_PER_G = 112
N_G = IDX_PER_STEP // IDX_PER_G
IDX_PER_W = B_PER_W * SP
NBUF = 2

BB1 = 128
BB2 = 128


def _sc_gather(idx_pad, table_pad):
    mesh = plsc.VectorSubcoreMesh(core_axis_name="c", subcore_axis_name="s")

    @functools.partial(
        pl.kernel,
        mesh=mesh,
        out_type=jax.ShapeDtypeStruct((B_SC, SP, DP), jnp.float32),
        scratch_types=[
            pltpu.VMEM_SHARED((V, DP), jnp.float32),
            pltpu.VMEM((IDX_PER_W,), jnp.int32),
            pltpu.VMEM((NBUF * NB * SP, DP), jnp.float32),
            pltpu.SemaphoreType.DMA,
            pltpu.SemaphoreType.DMA,
        ],
    )
    def k(idx_hbm, tab_hbm, out_hbm, tab_sp, idx_v, rows_v, sem_g, sem_w):
        cid = lax.axis_index("c")
        sid = lax.axis_index("s")
        wid = sid * NC + cid
        b0 = wid * B_PER_W

        @pl.when(sid == 0)
        def _():
            pltpu.sync_copy(tab_hbm, tab_sp)

        plsc.subcore_barrier()

        pltpu.sync_copy(idx_hbm.at[pl.ds(wid * IDX_PER_W, IDX_PER_W)], idx_v)
        rows_3d = rows_v.reshape(NBUF * NB, SP, DP)

        def step(i, carry):
            buf = i & 1

            @pl.when(i >= NBUF)
            def _():
                pltpu.make_async_copy(
                    rows_3d.at[pl.ds(0, NB)],
                    out_hbm.at[pl.ds(b0, NB)],
                    sem_w,
                ).wait()

            copies = []
            for g in range(N_G):
                idx_sl = idx_v.at[pl.ds(i * IDX_PER_STEP + g * IDX_PER_G,
                                        IDX_PER_G)]
                dst = rows_v.at[pl.ds(buf * NB * SP + g * IDX_PER_G,
                                      IDX_PER_G)]
                copies.append(pltpu.async_copy(tab_sp.at[idx_sl], dst, sem_g))
            for c in copies:
                c.wait()
            pltpu.async_copy(
                rows_3d.at[pl.ds(buf * NB, NB)],
                out_hbm.at[pl.ds(b0 + i * NB, NB)],
                sem_w,
            )
            return carry

        lax.fori_loop(0, N_STEPS, step, 0)

        for _ in range(NBUF):
            pltpu.make_async_copy(
                rows_3d.at[pl.ds(0, NB)],
                out_hbm.at[pl.ds(b0, NB)],
                sem_w,
            ).wait()

    return k(idx_pad, table_pad)


def _tc_matmul_body(sp_ref, tab_ref, out_ref):
    idx = sp_ref[...]
    onehot = (idx[:, :, None] ==
              lax.broadcasted_iota(jnp.int32, (BB1, S_COLS, V), 2))
    out_ref[...] = lax.dot_general(
        onehot.astype(jnp.float32), tab_ref[...],
        (((2,), (0,)), ((), ())),
        preferred_element_type=jnp.float32,
    )


def _tc_matmul(species_tc, table):
    return pl.pallas_call(
        _tc_matmul_body,
        grid=(A_TC // BB1,),
        in_specs=[
            pl.BlockSpec((BB1, S_COLS), lambda i: (i, 0)),
            pl.BlockSpec((V, D), lambda i: (0, 0)),
        ],
        out_specs=pl.BlockSpec((BB1, S_COLS, D), lambda i: (i, 0, 0)),
        out_shape=jax.ShapeDtypeStruct((B_ROWS, S_COLS, D), jnp.float32),
    )(species_tc, table)


def _tc_compact_body(full_ref, p_ref, out_ref):
    del full_ref
    out_ref[...] = p_ref[:, :S_COLS, :D]


def _tc_compact(full, padded):
    return pl.pallas_call(
        _tc_compact_body,
        grid=(B_SC // BB2,),
        in_specs=[
            pl.BlockSpec(memory_space=pl.ANY),
            pl.BlockSpec((BB2, SP, DP), lambda i: (i, 0, 0)),
        ],
        out_specs=pl.BlockSpec((BB2, S_COLS, D),
                               lambda i: (i + A_TC // BB2, 0, 0)),
        out_shape=jax.ShapeDtypeStruct((B_ROWS, S_COLS, D), jnp.float32),
        input_output_aliases={0: 0},
    )(full, padded)


def kernel(species, table):
    idx_pad = jnp.pad(species[A_TC:], ((0, 0), (0, SP - S_COLS)), mode="edge")
    table_pad = jnp.pad(table, ((0, 0), (0, DP - D)))
    padded = _sc_gather(idx_pad.reshape(B_SC * SP), table_pad)
    full = _tc_matmul(species[:A_TC], table)
    return _tc_compact(full, padded)


# speedup vs baseline: 1.4338x; 1.4338x over previous
import functools

import jax
import jax.numpy as jnp
from jax import lax
from jax.experimental import pallas as pl
from jax.experimental.pallas import tpu as pltpu
from jax.experimental.pallas import tpu_sc as plsc

B_ROWS = 4096
S_COLS = 50
SP = 56
D = 110
DP = 128
V = 96

NC = 2
NS = 16
NW = NC * NS
B_PER_W = B_ROWS // NW
NB = 4
N_STEPS = B_PER_W // NB
IDX_PER_STEP = NB * SP
IDX_PER_G = 112
N_G = IDX_PER_STEP // IDX_PER_G
IDX_PER_W = B_PER_W * SP
NBUF = 3


def _sc_gather(idx_pad, table_pad):
    mesh = plsc.VectorSubcoreMesh(core_axis_name="c", subcore_axis_name="s")

    @functools.partial(
        pl.kernel,
        mesh=mesh,
        out_type=jax.ShapeDtypeStruct((B_ROWS, SP, DP), jnp.float32),
        scratch_types=[
            pltpu.VMEM_SHARED((V, DP), jnp.float32),
            pltpu.VMEM((IDX_PER_W,), jnp.int32),
            pltpu.VMEM((NBUF * NB * SP, DP), jnp.float32),
            pltpu.SemaphoreType.DMA,
            pltpu.SemaphoreType.DMA,
        ],
    )
    def k(idx_hbm, tab_hbm, out_hbm, tab_sp, idx_v, rows_v, sem_g, sem_w):
        cid = lax.axis_index("c")
        sid = lax.axis_index("s")
        wid = sid * NC + cid
        b0 = wid * B_PER_W

        @pl.when(sid == 0)
        def _():
            pltpu.sync_copy(tab_hbm, tab_sp)

        plsc.subcore_barrier()

        pltpu.sync_copy(idx_hbm.at[pl.ds(wid * IDX_PER_W, IDX_PER_W)], idx_v)
        rows_3d = rows_v.reshape(NBUF * NB, SP, DP)

        def step(i, carry):
            buf = lax.rem(i, NBUF)

            @pl.when(i >= NBUF)
            def _():
                pltpu.make_async_copy(
                    rows_3d.at[pl.ds(0, NB)],
                    out_hbm.at[pl.ds(b0, NB)],
                    sem_w,
                ).wait()

            copies = []
            for g in range(N_G):
                idx_sl = idx_v.at[pl.ds(i * IDX_PER_STEP + g * IDX_PER_G,
                                        IDX_PER_G)]
                dst = rows_v.at[pl.ds(buf * NB * SP + g * IDX_PER_G,
                                      IDX_PER_G)]
                copies.append(pltpu.async_copy(tab_sp.at[idx_sl], dst, sem_g))
            for c in copies:
                c.wait()
            pltpu.async_copy(
                rows_3d.at[pl.ds(buf * NB, NB)],
                out_hbm.at[pl.ds(b0 + i * NB, NB)],
                sem_w,
            )
            return carry

        lax.fori_loop(0, N_STEPS, step, 0)

        for _ in range(NBUF):
            pltpu.make_async_copy(
                rows_3d.at[pl.ds(0, NB)],
                out_hbm.at[pl.ds(b0, NB)],
                sem_w,
            ).wait()

    return k(idx_pad, table_pad)


def kernel(species, table):
    idx_pad = jnp.pad(species, ((0, 0), (0, SP - S_COLS)), mode="edge")
    table_pad = jnp.pad(table, ((0, 0), (0, DP - D)))
    padded = _sc_gather(idx_pad.reshape(B_ROWS * SP), table_pad)
    return padded[:, :S_COLS, :D]
